# baseline (speedup 1.0000x reference)
import functools

import jax
import jax.numpy as jnp
from jax import lax
from jax.experimental import pallas as pl
from jax.experimental.pallas import tpu as pltpu
from jax.experimental.pallas import tpu_sc as plsc

B, L, H = 1024, 128, 128
BL = B * L
NF = 9
_GSIZES = (238, 192, 196, 144)
NG = len(_GSIZES)

NC, NS = 2, 16
NW = NC * NS
ROWS_PER_W = BL // NW
C = 64
NCHUNK = ROWS_PER_W // C
HV = H // 16


def _make_sc_call():
    mesh = plsc.VectorSubcoreMesh(
        core_axis_name="c", subcore_axis_name="s", num_cores=NC, num_subcores=NS
    )
    scratch = (
        [pltpu.VMEM((n * H,), jnp.float32) for n in _GSIZES]
        + [pltpu.VMEM((3 * H,), jnp.float32)]
        + [pltpu.VMEM((NF * C,), jnp.int32) for _ in range(2)]
        + [pltpu.VMEM((C * 3,), jnp.float32) for _ in range(2)]
        + [pltpu.VMEM((C * H,), jnp.float32) for _ in range(2)]
        + [pltpu.SemaphoreType.DMA for _ in range(4)]
    )

    @functools.partial(
        pl.kernel,
        mesh=mesh,
        out_type=jax.ShapeDtypeStruct((BL * H,), jnp.float32),
        scratch_types=scratch,
    )
    def sc_kernel(idx_hbm, tab0, tab1, tab2, tab3, pos_hbm, wp_hbm, out_hbm,
                  tv0, tv1, tv2, tv3, wp_v, ix0, ix1, pv0, pv1, ov0, ov1,
                  si0, si1, so0, so1):
        tab_hbm = (tab0, tab1, tab2, tab3)
        tab_v = (tv0, tv1, tv2, tv3)
        idx_v = (ix0, ix1)
        pos_v = (pv0, pv1)
        out_v = (ov0, ov1)
        sin = (si0, si1)
        sout = (so0, so1)

        wid = lax.axis_index("s") * NC + lax.axis_index("c")
        base0 = wid * ROWS_PER_W

        for g in range(NG):
            pltpu.sync_copy(tab_hbm[g], tab_v[g])
        pltpu.sync_copy(wp_hbm, wp_v)

        def issue_in(chunk, slot):
            base = base0 + chunk * C
            pltpu.async_copy(idx_hbm.at[pl.ds(base * NF, NF * C)], idx_v[slot],
                             sin[slot])
            pltpu.async_copy(pos_hbm.at[pl.ds(base * 3, C * 3)], pos_v[slot],
                             sin[slot])

        def wait_in(chunk, slot):
            base = base0 + chunk * C
            pltpu.make_async_copy(idx_hbm.at[pl.ds(base * NF, NF * C)],
                                  idx_v[slot], sin[slot]).wait()
            pltpu.make_async_copy(pos_hbm.at[pl.ds(base * 3, C * 3)], pos_v[slot],
                                  sin[slot]).wait()

        def out_descr(chunk, slot):
            base = base0 + chunk * C
            return pltpu.make_async_copy(
                out_v[slot], out_hbm.at[pl.ds(base * H, C * H)], sout[slot])

        issue_in(0, 0)
        issue_in(1, 1)

        def compute_chunk(chunk, slot):
            wp_vecs = tuple(
                wp_v[pl.ds(k * H + j * 16, 16)] for k in range(3) for j in range(HV)
            )

            def group_body(g, wp_c):
                iv = [idx_v[slot][pl.ds((g * 16 * NF) + f * 16, 16)]
                      for f in range(NF)]
                cv = [
                    iv[0] * 2 + iv[5],
                    iv[1] * 12 + iv[2],
                    iv[3] * 14 + iv[4],
                    (iv[6] * 9 + iv[7]) * 2 + iv[8],
                ]
                pvecs = [pos_v[slot][pl.ds(g * 48 + m * 16, 16)] for m in range(3)]
                for rr in range(16):
                    idx = [cv[t][rr] for t in range(NG)]
                    pv = [
                        jnp.full((16,),
                                 pvecs[(rr * 3 + k) // 16][(rr * 3 + k) % 16],
                                 jnp.float32)
                        for k in range(3)
                    ]
                    rowoff = (g * 16 + rr) * H
                    for j in range(HV):
                        t01 = (tab_v[0][pl.ds(idx[0] * H + j * 16, 16)]
                               + tab_v[1][pl.ds(idx[1] * H + j * 16, 16)])
                        t23 = (tab_v[2][pl.ds(idx[2] * H + j * 16, 16)]
                               + tab_v[3][pl.ds(idx[3] * H + j * 16, 16)])
                        pacc = (pv[0] * wp_c[j] + pv[1] * wp_c[HV + j]
                                + pv[2] * wp_c[2 * HV + j])
                        out_v[slot][pl.ds(rowoff + j * 16, 16)] = (t01 + t23) + pacc
                return wp_c

            lax.fori_loop(0, C // 16, group_body, wp_vecs)

        def body2(i2, carry):
            for slot in range(2):
                chunk = i2 * 2 + slot
                wait_in(chunk, slot)

                @pl.when(i2 > 0)
                def _():
                    out_descr(chunk - 2, slot).wait()

                compute_chunk(chunk, slot)
                out_descr(chunk, slot).start()

                @pl.when(i2 + 1 < NCHUNK // 2)
                def _():
                    issue_in(chunk + 2, slot)

            return carry

        lax.fori_loop(0, NCHUNK // 2, body2, 0)
        out_descr(NCHUNK - 2, 0).wait()
        out_descr(NCHUNK - 1, 1).wait()

    return sc_kernel


_SC_CALL = _make_sc_call()


def kernel(atomic_number, formal_charge, degree, explicit_valence,
           implicit_valence, aromatic, hybridization, total_num_H, is_in_ring,
           W_atomic_number, W_formal_charge, W_degree, W_explicit_valence,
           W_implicit_valence, W_aromatic, W_hybridization, W_total_num_H,
           W_is_in_ring, position, W_position):
    idxs = [atomic_number, formal_charge, degree, explicit_valence,
            implicit_valence, aromatic, hybridization, total_num_H, is_in_ring]
    idx = jnp.stack([i.reshape(BL).astype(jnp.int32) for i in idxs])
    idx = (idx.reshape(NF, NW * NCHUNK * (C // 16), 16)
           .transpose(1, 0, 2).reshape(-1))
    f32 = jnp.float32
    g0 = (W_atomic_number.astype(f32)[:, None, :]
          + W_aromatic.astype(f32)[None, :, :]).reshape(-1)
    g1 = (W_formal_charge.astype(f32)[:, None, :]
          + W_degree.astype(f32)[None, :, :]).reshape(-1)
    g2 = (W_explicit_valence.astype(f32)[:, None, :]
          + W_implicit_valence.astype(f32)[None, :, :]).reshape(-1)
    g3 = (W_hybridization.astype(f32)[:, None, None, :]
          + W_total_num_H.astype(f32)[None, :, None, :]
          + W_is_in_ring.astype(f32)[None, None, :, :]).reshape(-1)
    pos = position.reshape(BL * 3).astype(f32)
    wp = W_position.reshape(3 * H).astype(f32)
    out = _SC_CALL(idx, g0, g1, g2, g3, pos, wp)
    return out.reshape(B, L, H)


# speedup vs baseline: 1.2252x; 1.2252x over previous
import functools

import jax
import jax.numpy as jnp
from jax import lax
from jax.experimental import pallas as pl
from jax.experimental.pallas import tpu as pltpu
from jax.experimental.pallas import tpu_sc as plsc

B, L, H = 1024, 128, 128
BL = B * L
NF = 9
_GSIZES = (238, 192, 196, 144)
NG = len(_GSIZES)

NC, NS = 2, 16
NW = NC * NS
ROWS_PER_W = BL // NW
C = 128
NCHUNK = ROWS_PER_W // C
HV = H // 16


def _make_sc_call():
    mesh = plsc.VectorSubcoreMesh(
        core_axis_name="c", subcore_axis_name="s", num_cores=NC, num_subcores=NS
    )
    scratch = (
        [pltpu.VMEM((n * H,), jnp.float32) for n in _GSIZES]
        + [pltpu.VMEM((3 * H,), jnp.float32)]
        + [pltpu.VMEM((NF * C,), jnp.int32)]
        + [pltpu.VMEM((C * 3,), jnp.float32)]
        + [pltpu.VMEM((C * H,), jnp.float32)]
    )

    @functools.partial(
        pl.kernel,
        mesh=mesh,
        out_type=jax.ShapeDtypeStruct((BL * H,), jnp.float32),
        scratch_types=scratch,
    )
    def sc_kernel(idx_hbm, tab0, tab1, tab2, tab3, pos_hbm, wp_hbm, out_hbm,
                  tv0, tv1, tv2, tv3, wp_v, idx_v, pos_v, out_v):
        tab_hbm = (tab0, tab1, tab2, tab3)
        tab_v = (tv0, tv1, tv2, tv3)

        wid = lax.axis_index("s") * NC + lax.axis_index("c")
        base0 = wid * ROWS_PER_W

        for g in range(NG):
            pltpu.sync_copy(tab_hbm[g], tab_v[g])
        pltpu.sync_copy(wp_hbm, wp_v)

        def chunk_body(it, carry_outer):
            base = base0 + it * C
            pltpu.sync_copy(idx_hbm.at[pl.ds(base * NF, NF * C)], idx_v)
            pltpu.sync_copy(pos_hbm.at[pl.ds(base * 3, C * 3)], pos_v)

            wp_vecs = tuple(
                wp_v[pl.ds(k * H + j * 16, 16)] for k in range(3) for j in range(HV)
            )

            @plsc.parallel_loop(0, C // 16, carry=wp_vecs)
            def group_body(g, wp_c):
                iv = [idx_v[pl.ds((g * 16 * NF) + f * 16, 16)]
                      for f in range(NF)]
                cv = [
                    iv[0] * 2 + iv[5],
                    iv[1] * 12 + iv[2],
                    iv[3] * 14 + iv[4],
                    (iv[6] * 9 + iv[7]) * 2 + iv[8],
                ]
                pvecs = [pos_v[pl.ds(g * 48 + m * 16, 16)] for m in range(3)]
                for rr in range(16):
                    idx = [cv[t][rr] for t in range(NG)]
                    pv = [
                        jnp.full((16,),
                                 pvecs[(rr * 3 + k) // 16][(rr * 3 + k) % 16],
                                 jnp.float32)
                        for k in range(3)
                    ]
                    rowoff = (g * 16 + rr) * H
                    for j in range(HV):
                        t01 = (tab_v[0][pl.ds(idx[0] * H + j * 16, 16)]
                               + tab_v[1][pl.ds(idx[1] * H + j * 16, 16)])
                        t23 = (tab_v[2][pl.ds(idx[2] * H + j * 16, 16)]
                               + tab_v[3][pl.ds(idx[3] * H + j * 16, 16)])
                        pacc = (pv[0] * wp_c[j] + pv[1] * wp_c[HV + j]
                                + pv[2] * wp_c[2 * HV + j])
                        out_v[pl.ds(rowoff + j * 16, 16)] = (t01 + t23) + pacc
                return wp_c

            pltpu.sync_copy(out_v, out_hbm.at[pl.ds(base * H, C * H)])
            return carry_outer

        lax.fori_loop(0, NCHUNK, chunk_body, 0)

    return sc_kernel


_SC_CALL = _make_sc_call()


def kernel(atomic_number, formal_charge, degree, explicit_valence,
           implicit_valence, aromatic, hybridization, total_num_H, is_in_ring,
           W_atomic_number, W_formal_charge, W_degree, W_explicit_valence,
           W_implicit_valence, W_aromatic, W_hybridization, W_total_num_H,
           W_is_in_ring, position, W_position):
    idxs = [atomic_number, formal_charge, degree, explicit_valence,
            implicit_valence, aromatic, hybridization, total_num_H, is_in_ring]
    idx = jnp.stack([i.reshape(BL).astype(jnp.int32) for i in idxs])
    idx = (idx.reshape(NF, NW * NCHUNK * (C // 16), 16)
           .transpose(1, 0, 2).reshape(-1))
    f32 = jnp.float32
    g0 = (W_atomic_number.astype(f32)[:, None, :]
          + W_aromatic.astype(f32)[None, :, :]).reshape(-1)
    g1 = (W_formal_charge.astype(f32)[:, None, :]
          + W_degree.astype(f32)[None, :, :]).reshape(-1)
    g2 = (W_explicit_valence.astype(f32)[:, None, :]
          + W_implicit_valence.astype(f32)[None, :, :]).reshape(-1)
    g3 = (W_hybridization.astype(f32)[:, None, None, :]
          + W_total_num_H.astype(f32)[None, :, None, :]
          + W_is_in_ring.astype(f32)[None, None, :, :]).reshape(-1)
    pos = position.reshape(BL * 3).astype(f32)
    wp = W_position.reshape(3 * H).astype(f32)
    out = _SC_CALL(idx, g0, g1, g2, g3, pos, wp)
    return out.reshape(B, L, H)
